# baseline (speedup 1.0000x reference)
import functools

import jax
import jax.numpy as jnp
import numpy as np
from jax import lax
from jax.experimental import pallas as pl
from jax.experimental.pallas import tpu as pltpu
from jax.experimental.pallas import tpu_sc as plsc

VOCAB = 1000000
EMB = 64
OUT = 128
B = 16384
L = 200

NC = 2
NS = 16
NW = NC * NS
BLK = 128
KPW = B // (NW * BLK)
NBUF = 2
GRP = 8
NGRP = L // GRP
LPAD = L + GRP

_mesh = plsc.VectorSubcoreMesh(
    core_axis_name="c", subcore_axis_name="s", num_cores=NC, num_subcores=NS
)


@functools.partial(
    pl.kernel,
    out_type=jax.ShapeDtypeStruct((B, EMB), jnp.float32),
    mesh=_mesh,
    scratch_types=[
        pltpu.VMEM((LPAD * BLK,), jnp.int32),
        pltpu.VMEM((NBUF, GRP * BLK, EMB), jnp.bfloat16),
        pltpu.VMEM((BLK, EMB), jnp.float32),
        [pltpu.SemaphoreType.DMA] * NBUF,
    ],
    compiler_params=pltpu.CompilerParams(
        use_tc_tiling_on_sc=False, needs_layout_passes=False
    ),
)
def _pooled_sums(xt_hbm, table_hbm, out_hbm, idx_v, rows_v, accum_v, sems):
    wid = lax.axis_index("s") * NC + lax.axis_index("c")

    def _accumulate_group(bb):
        def _slab(s, c1):
            def _acc(i, c2):
                r = s * BLK + i
                for gg in range(EMB // 32):
                    packed = rows_v[bb, r, pl.ds(gg * 32, 32)]
                    lo, hi = plsc.unpack(
                        packed,
                        format=plsc.PackFormat.INTERLEAVED,
                        preferred_element_type=jnp.float32,
                    )
                    plsc.addupdate(accum_v.at[i, pl.ds(gg * 32, 16)], lo)
                    plsc.addupdate(accum_v.at[i, pl.ds(gg * 32 + 16, 16)], hi)
                return c2

            lax.fori_loop(0, BLK, _acc, 0, unroll=8)
            return c1

        lax.fori_loop(0, GRP, _slab, 0)

    def _block(k, carry):
        blk = wid * KPW + k
        pltpu.sync_copy(xt_hbm.at[blk], idx_v)

        for bb in range(NBUF):
            pltpu.async_copy(
                table_hbm.at[idx_v.at[pl.ds(bb * GRP * BLK, GRP * BLK)]],
                rows_v.at[bb],
                sems[bb],
            )

        @plsc.parallel_loop(0, BLK, 1, unroll=8)
        def _zero(i):
            zero = jnp.zeros((16,), jnp.float32)
            for cc in range(EMB // 16):
                accum_v[i, pl.ds(cc * 16, 16)] = zero

        def _group(m, carry):
            for bb in range(NBUF):
                g = m * NBUF + bb
                pltpu.make_async_copy(
                    table_hbm.at[pl.ds(0, GRP * BLK)], rows_v.at[bb], sems[bb]
                ).wait()

                _accumulate_group(bb)

                pltpu.async_copy(
                    table_hbm.at[
                        idx_v.at[pl.ds((g + NBUF) * GRP * BLK, GRP * BLK)]
                    ],
                    rows_v.at[bb],
                    sems[bb],
                )
            return carry

        lax.fori_loop(0, (NGRP - 1) // NBUF, _group, 0)

        pltpu.make_async_copy(
            table_hbm.at[pl.ds(0, GRP * BLK)], rows_v.at[0], sems[0]
        ).wait()
        _accumulate_group(0)
        pltpu.make_async_copy(
            table_hbm.at[pl.ds(0, GRP * BLK)], rows_v.at[1], sems[1]
        ).wait()

        pltpu.sync_copy(accum_v, out_hbm.at[pl.ds(blk * BLK, BLK)])
        return carry

    lax.fori_loop(0, KPW, _block, 0)


def _project(pooled, W, b):
    BS = 1024

    def body(p_ref, w_ref, b_ref, o_ref):
        o_ref[...] = (
            jnp.dot(p_ref[...], w_ref[...], preferred_element_type=jnp.float32)
            * (1.0 / L)
            + b_ref[...]
        )

    return pl.pallas_call(
        body,
        grid=(B // BS,),
        in_specs=[
            pl.BlockSpec((BS, EMB), lambda i: (i, 0)),
            pl.BlockSpec((EMB, OUT), lambda i: (0, 0)),
            pl.BlockSpec((1, OUT), lambda i: (0, 0)),
        ],
        out_specs=pl.BlockSpec((BS, OUT), lambda i: (i, 0)),
        out_shape=jax.ShapeDtypeStruct((B, OUT), jnp.float32),
    )(pooled, W, b.reshape(1, OUT))


_PERM = np.array(
    sum(
        (
            [g * 32 + 2 * t for t in range(16)]
            + [g * 32 + 2 * t + 1 for t in range(16)]
            for g in range(EMB // 32)
        ),
        [],
    ),
    dtype=np.int32,
)


def kernel(x, table, W, b):
    x32 = x.astype(jnp.int32)
    xt = x32.reshape(NW * KPW, BLK, L).transpose(0, 2, 1)
    xt = jnp.pad(xt, ((0, 0), (0, LPAD - L), (0, 0)))
    xt = xt.reshape(NW * KPW, LPAD * BLK)
    table_bf = table.astype(jnp.bfloat16)
    pooled = _pooled_sums(xt, table_bf)
    return _project(pooled, W[_PERM], b)


# speedup vs baseline: 2.2609x; 2.2609x over previous
import functools

import jax
import jax.numpy as jnp
from jax import lax
from jax.experimental import pallas as pl
from jax.experimental.pallas import tpu as pltpu
from jax.experimental.pallas import tpu_sc as plsc

VOCAB = 1000000
EMB = 64
OUT = 128
B = 16384
L = 200

NC = 2
NS = 16
NW = NC * NS
BLK = 128
KPW = B // (NW * BLK)
NBUF = 2

_mesh = plsc.VectorSubcoreMesh(
    core_axis_name="c", subcore_axis_name="s", num_cores=NC, num_subcores=NS
)


@functools.partial(
    pl.kernel,
    out_type=jax.ShapeDtypeStruct((B, EMB), jnp.float32),
    mesh=_mesh,
    scratch_types=[
        pltpu.VMEM((BLK * L,), jnp.int32),
        pltpu.VMEM((NBUF, BLK), jnp.int32),
        pltpu.VMEM((NBUF, BLK, EMB), jnp.float32),
        pltpu.VMEM((BLK, EMB), jnp.float32),
        [pltpu.SemaphoreType.DMA] * NBUF,
    ],
    compiler_params=pltpu.CompilerParams(
        use_tc_tiling_on_sc=False, needs_layout_passes=False
    ),
)
def _pooled_sums(x_hbm, table_hbm, out_hbm, xv, idx_v, rows_v, accum_v, sems):
    wid = lax.axis_index("s") * NC + lax.axis_index("c")

    def _build_idx(bb, l):
        lane = lax.iota(jnp.int32, 16) * L + l
        for c in range(BLK // 16):
            vals = plsc.load_gather(xv, [lane + c * 16 * L])
            idx_v[bb, pl.ds(c * 16, 16)] = vals

    def _block(k, carry):
        blk = wid * KPW + k
        pltpu.sync_copy(x_hbm.at[pl.ds(blk * BLK * L, BLK * L)], xv)

        for bb in range(NBUF):
            _build_idx(bb, bb)
            pltpu.async_copy(table_hbm.at[idx_v.at[bb]], rows_v.at[bb], sems[bb])

        @plsc.parallel_loop(0, BLK, 1, unroll=8)
        def _zero(i):
            zero = jnp.zeros((16,), jnp.float32)
            for cc in range(EMB // 16):
                accum_v[i, pl.ds(cc * 16, 16)] = zero

        def _acc_slab(bb_rows):
            def _acc(i, c2):
                for cc in range(EMB // 16):
                    plsc.addupdate(
                        accum_v.at[i, pl.ds(cc * 16, 16)],
                        bb_rows[i, pl.ds(cc * 16, 16)],
                    )
                return c2

            lax.fori_loop(0, BLK, _acc, 0, unroll=8)

        def _step(m, carry2):
            for bb in range(NBUF):
                l = m * NBUF + bb
                pltpu.make_async_copy(
                    table_hbm.at[pl.ds(0, BLK)], rows_v.at[bb], sems[bb]
                ).wait()
                _acc_slab(rows_v.at[bb])
                l_next = jnp.minimum(l + NBUF, L - 1)
                _build_idx(bb, l_next)
                pltpu.async_copy(
                    table_hbm.at[idx_v.at[bb]], rows_v.at[bb], sems[bb]
                )
            return carry2

        lax.fori_loop(0, L // NBUF, _step, 0)

        for bb in range(NBUF):
            pltpu.make_async_copy(
                table_hbm.at[pl.ds(0, BLK)], rows_v.at[bb], sems[bb]
            ).wait()

        pltpu.sync_copy(accum_v, out_hbm.at[pl.ds(blk * BLK, BLK)])
        return carry

    lax.fori_loop(0, KPW, _block, 0)


def _project(pooled, W, b):
    BS = 1024

    def body(p_ref, w_ref, b_ref, o_ref):
        o_ref[...] = (
            jnp.dot(p_ref[...], w_ref[...], preferred_element_type=jnp.float32)
            * (1.0 / L)
            + b_ref[...]
        )

    return pl.pallas_call(
        body,
        grid=(B // BS,),
        in_specs=[
            pl.BlockSpec((BS, EMB), lambda i: (i, 0)),
            pl.BlockSpec((EMB, OUT), lambda i: (0, 0)),
            pl.BlockSpec((1, OUT), lambda i: (0, 0)),
        ],
        out_specs=pl.BlockSpec((BS, OUT), lambda i: (i, 0)),
        out_shape=jax.ShapeDtypeStruct((B, OUT), jnp.float32),
    )(pooled, W, b.reshape(1, OUT))


def kernel(x, table, W, b):
    x_flat = x.astype(jnp.int32).reshape(B * L)
    pooled = _pooled_sums(x_flat, table)
    return _project(pooled, W, b)
